# baseline (speedup 1.0000x reference)
import functools

import jax
import jax.numpy as jnp
from jax import lax
from jax.experimental import pallas as pl
from jax.experimental.pallas import tpu as pltpu
from jax.experimental.pallas import tpu_sc as plsc

NC = 2
NS = 16
NW = NC * NS
K = 80


def _deg_kernel(E, NP):
    ept = E // NW
    nit = ept // K
    sl = NP // NS
    mesh = plsc.VectorSubcoreMesh(core_axis_name="c", subcore_axis_name="s")

    @functools.partial(
        pl.kernel,
        mesh=mesh,
        out_type=jax.ShapeDtypeStruct((NS, NC, NP // NS), jnp.float32),
        scratch_types=[
            pltpu.VMEM((nit, K), jnp.int32),
            pltpu.VMEM((K,), jnp.float32),
            pltpu.VMEM_SHARED((NP,), jnp.float32),
            pltpu.SemaphoreType.DMA,
        ],
    )
    def deg_k(row_hbm, zeros_hbm, out_hbm, idx_v, ones_v, deg_sp, sem):
        c = lax.axis_index("c")
        s = lax.axis_index("s")
        w = c * NS + s
        pltpu.sync_copy(row_hbm.at[w], idx_v)
        pltpu.sync_copy(zeros_hbm, deg_sp.at[pl.ds(s * sl, sl)])
        for i in range(K // 16):
            ones_v[pl.ds(i * 16, 16)] = jnp.full((16,), 1.0, jnp.float32)
        plsc.subcore_barrier()

        pltpu.async_copy(ones_v, deg_sp.at[idx_v.at[0]], sem, add=True)

        def body(it, carry):
            pltpu.async_copy(ones_v, deg_sp.at[idx_v.at[it + 1]], sem, add=True)
            pltpu.make_async_copy(ones_v, deg_sp.at[idx_v.at[it]], sem).wait()
            return carry

        lax.fori_loop(0, nit - 1, body, 0)
        pltpu.make_async_copy(ones_v, deg_sp.at[idx_v.at[nit - 1]], sem).wait()
        plsc.subcore_barrier()
        pltpu.sync_copy(deg_sp.at[pl.ds(s * sl, sl)], out_hbm.at[s, c])

    return deg_k


def _agg_kernel(E, N, NP, D):
    ept = E // NW
    nit = ept // K
    sl = NP // NS
    mesh = plsc.VectorSubcoreMesh(core_axis_name="c", subcore_axis_name="s")

    assert nit % 2 == 1

    @functools.partial(
        pl.kernel,
        mesh=mesh,
        out_type=jax.ShapeDtypeStruct((NC, NP, D), jnp.float32),
        scratch_types=[
            pltpu.VMEM((2, K), jnp.int32),
            pltpu.VMEM((2, K), jnp.int32),
            pltpu.VMEM((K, D), jnp.float32),
            pltpu.VMEM((K, D), jnp.float32),
            pltpu.VMEM_SHARED((NP, D), jnp.float32),
            pltpu.SemaphoreType.DMA,
            pltpu.SemaphoreType.DMA,
        ],
    )
    def agg_k(cr_hbm, xs_hbm, zeros_hbm, out_hbm,
              cr0, cr1, m0, m1, acc_sp, semA, semB):
        c = lax.axis_index("c")
        s = lax.axis_index("s")
        w = c * NS + s
        pltpu.sync_copy(zeros_hbm, acc_sp.at[pl.ds(s * sl, sl)])
        plsc.subcore_barrier()

        def load_idx(it, crbuf):
            pltpu.sync_copy(cr_hbm.at[w, it], crbuf)

        def gather(crbuf, buf, sem):
            pltpu.async_copy(xs_hbm.at[crbuf.at[1]], buf, sem)

        def gwait(crbuf, buf, sem):
            pltpu.make_async_copy(xs_hbm.at[crbuf.at[1]], buf, sem).wait()

        def scat(crbuf, buf):
            pltpu.sync_copy(buf, acc_sp.at[crbuf.at[0]], add=True)

        load_idx(0, cr0)
        gather(cr0, m0, semA)
        load_idx(1, cr1)
        gather(cr1, m1, semB)

        def body(j, carry):
            a = 2 * j
            gwait(cr0, m0, semA)
            scat(cr0, m0)
            load_idx(a + 2, cr0)
            gather(cr0, m0, semA)
            gwait(cr1, m1, semB)
            scat(cr1, m1)

            @pl.when(a + 3 < nit)
            def _():
                load_idx(a + 3, cr1)
                gather(cr1, m1, semB)

            return carry

        lax.fori_loop(0, (nit - 1) // 2, body, 0)
        gwait(cr0, m0, semA)
        scat(cr0, m0)
        plsc.subcore_barrier()
        pltpu.sync_copy(acc_sp.at[pl.ds(s * sl, sl)],
                        out_hbm.at[c, pl.ds(s * sl, sl)])

    return agg_k


def _scale_body(dp_ref, v_ref, o_ref):
    dp = dp_ref[0]
    deg = dp[0] + dp[1]
    dinv = jnp.where(deg > 0.0, lax.rsqrt(deg), 0.0)
    v = v_ref[...]
    agg = v[0] if v.shape[0] == 1 else v[0] + v[1]
    o_ref[...] = agg * dinv[:, None]


def _scale_call(dp3, v, n_out, d):
    nb, _, rb = dp3.shape
    p = v.shape[0]
    return pl.pallas_call(
        _scale_body,
        grid=(nb,),
        in_specs=[
            pl.BlockSpec((1, 2, rb), lambda i: (i, 0, 0)),
            pl.BlockSpec((p, rb, d), lambda i: (0, i, 0)),
        ],
        out_specs=pl.BlockSpec((rb, d), lambda i: (i, 0)),
        out_shape=jax.ShapeDtypeStruct((n_out, d), jnp.float32),
    )(dp3, v)


@jax.jit
def kernel(x, edge_index):
    N, D = x.shape
    E = edge_index.shape[1]
    NP = 10240
    ept = E // NW
    nit = ept // K
    row3 = edge_index[0].reshape(NW, nit, K)
    cr4 = edge_index.reshape(2, NW, nit, K).transpose(1, 2, 0, 3)

    zeros1 = jnp.zeros((NP // NS,), jnp.float32)
    zeros2 = jnp.zeros((NP // NS, D), jnp.float32)

    dp3 = _deg_kernel(E, NP)(row3, zeros1)
    xs = _scale_call(dp3, x.reshape(1, N, D), NP, D)
    parts = _agg_kernel(E, N, NP, D)(cr4, xs, zeros2)
    out = _scale_call(dp3, parts, N, D)
    return out


# speedup vs baseline: 1.1682x; 1.1682x over previous
import functools

import jax
import jax.numpy as jnp
from jax import lax
from jax.experimental import pallas as pl
from jax.experimental.pallas import tpu as pltpu
from jax.experimental.pallas import tpu_sc as plsc

NC = 2
NS = 16
NW = NC * NS
K = 80


def _deg_kernel(E, NP):
    ept = E // NW
    nit = ept // K
    sl = NP // NS
    mesh = plsc.VectorSubcoreMesh(core_axis_name="c", subcore_axis_name="s")

    @functools.partial(
        pl.kernel,
        mesh=mesh,
        out_type=jax.ShapeDtypeStruct((NS, NC, NP // NS), jnp.float32),
        scratch_types=[
            pltpu.VMEM((nit, K), jnp.int32),
            pltpu.VMEM((K,), jnp.float32),
            pltpu.VMEM_SHARED((NP,), jnp.float32),
            pltpu.SemaphoreType.DMA,
        ],
    )
    def deg_k(row_hbm, zeros_hbm, out_hbm, idx_v, ones_v, deg_sp, sem):
        c = lax.axis_index("c")
        s = lax.axis_index("s")
        w = c * NS + s
        pltpu.sync_copy(row_hbm.at[w], idx_v)
        pltpu.sync_copy(zeros_hbm, deg_sp.at[pl.ds(s * sl, sl)])
        for i in range(K // 16):
            ones_v[pl.ds(i * 16, 16)] = jnp.full((16,), 1.0, jnp.float32)
        plsc.subcore_barrier()

        pltpu.async_copy(ones_v, deg_sp.at[idx_v.at[0]], sem, add=True)

        def body(it, carry):
            pltpu.async_copy(ones_v, deg_sp.at[idx_v.at[it + 1]], sem, add=True)
            pltpu.make_async_copy(ones_v, deg_sp.at[idx_v.at[it]], sem).wait()
            return carry

        lax.fori_loop(0, nit - 1, body, 0)
        pltpu.make_async_copy(ones_v, deg_sp.at[idx_v.at[nit - 1]], sem).wait()
        plsc.subcore_barrier()
        pltpu.sync_copy(deg_sp.at[pl.ds(s * sl, sl)], out_hbm.at[s, c])

    return deg_k


def _agg_kernel(E, N, NP, D):
    ept = E // NW
    nit = ept // K
    sl = NP // NS
    mesh = plsc.VectorSubcoreMesh(core_axis_name="c", subcore_axis_name="s")

    assert nit % 3 == 2

    @functools.partial(
        pl.kernel,
        mesh=mesh,
        out_type=jax.ShapeDtypeStruct((NC, NP, D), jnp.float32),
        scratch_types=[
            pltpu.VMEM((2, K), jnp.int32),
            pltpu.VMEM((2, K), jnp.int32),
            pltpu.VMEM((2, K), jnp.int32),
            pltpu.VMEM((K, D), jnp.float32),
            pltpu.VMEM((K, D), jnp.float32),
            pltpu.VMEM((K, D), jnp.float32),
            pltpu.VMEM_SHARED((NP, D), jnp.float32),
            pltpu.SemaphoreType.DMA,
            pltpu.SemaphoreType.DMA,
            pltpu.SemaphoreType.DMA,
            pltpu.SemaphoreType.DMA,
            pltpu.SemaphoreType.DMA,
            pltpu.SemaphoreType.DMA,
        ],
    )
    def agg_k(cr_hbm, xs_hbm, zeros_hbm, out_hbm,
              cr0, cr1, cr2, m0, m1, m2, acc_sp, g0, g1, g2, i0, i1, i2):
        c = lax.axis_index("c")
        s = lax.axis_index("s")
        w = c * NS + s
        pltpu.sync_copy(zeros_hbm, acc_sp.at[pl.ds(s * sl, sl)])
        plsc.subcore_barrier()

        B0 = (cr0, i0)
        B1 = (cr1, i1)
        B2 = (cr2, i2)
        M0 = (m0, g0)
        M1 = (m1, g1)
        M2 = (m2, g2)

        def iload(it, cr):
            pltpu.async_copy(cr_hbm.at[w, it], cr[0], cr[1])

        def iwait(it, cr):
            pltpu.make_async_copy(cr_hbm.at[w, it], cr[0], cr[1]).wait()

        def gather(it, cr, m):
            pltpu.async_copy(xs_hbm.at[cr[0].at[1]], m[0], m[1])

        def gwait(it, cr, m):
            pltpu.make_async_copy(xs_hbm.at[cr[0].at[1]], m[0], m[1]).wait()

        def scat(cr, m):
            pltpu.sync_copy(m[0], acc_sp.at[cr[0].at[0]], add=True)

        iload(0, B0)
        iwait(0, B0)
        gather(0, B0, M0)
        iload(1, B1)
        iload(2, B2)
        iwait(1, B1)
        gather(1, B1, M1)

        def body2(t, carry):
            a = 3 * t

            def sub(cc, B, M, Bn, Mn):
                gwait(cc, B, M)
                scat(B, M)

                @pl.when(cc + 3 < nit)
                def _():
                    iload(cc + 3, B)

                @pl.when(cc + 2 < nit)
                def _():
                    iwait(cc + 2, Bn)
                    gather(cc + 2, Bn, Mn)

            sub(a, B0, M0, B2, M2)
            sub(a + 1, B1, M1, B0, M0)
            sub(a + 2, B2, M2, B1, M1)
            return carry

        lax.fori_loop(0, (nit - 2) // 3, body2, 0)
        gwait(nit - 2, B0, M0)
        scat(B0, M0)
        gwait(nit - 1, B1, M1)
        scat(B1, M1)
        plsc.subcore_barrier()
        pltpu.sync_copy(acc_sp.at[pl.ds(s * sl, sl)],
                        out_hbm.at[c, pl.ds(s * sl, sl)])

    return agg_k


def _scale_body(dp_ref, v_ref, o_ref):
    dp = dp_ref[0]
    deg = dp[0] + dp[1]
    dinv = jnp.where(deg > 0.0, lax.rsqrt(deg), 0.0)
    v = v_ref[...]
    agg = v[0] if v.shape[0] == 1 else v[0] + v[1]
    o_ref[...] = agg * dinv[:, None]


def _scale_call(dp3, v, n_out, d):
    nb, _, rb = dp3.shape
    p = v.shape[0]
    return pl.pallas_call(
        _scale_body,
        grid=(nb,),
        in_specs=[
            pl.BlockSpec((1, 2, rb), lambda i: (i, 0, 0)),
            pl.BlockSpec((p, rb, d), lambda i: (0, i, 0)),
        ],
        out_specs=pl.BlockSpec((rb, d), lambda i: (i, 0)),
        out_shape=jax.ShapeDtypeStruct((n_out, d), jnp.float32),
    )(dp3, v)


@jax.jit
def kernel(x, edge_index):
    N, D = x.shape
    E = edge_index.shape[1]
    NP = 10240
    ept = E // NW
    nit = ept // K
    row3 = edge_index[0].reshape(NW, nit, K)
    cr4 = edge_index.reshape(2, NW, nit, K).transpose(1, 2, 0, 3)

    zeros1 = jnp.zeros((NP // NS,), jnp.float32)
    zeros2 = jnp.zeros((NP // NS, D), jnp.float32)

    dp3 = _deg_kernel(E, NP)(row3, zeros1)
    xs = _scale_call(dp3, x.reshape(1, N, D), NP, D)
    parts = _agg_kernel(E, N, NP, D)(cr4, xs, zeros2)
    out = _scale_call(dp3, parts, N, D)
    return out


# speedup vs baseline: 1.2830x; 1.0983x over previous
import functools

import jax
import jax.numpy as jnp
from jax import lax
from jax.experimental import pallas as pl
from jax.experimental.pallas import tpu as pltpu
from jax.experimental.pallas import tpu_sc as plsc

NC = 2
NS = 16
NW = NC * NS
K = 80


def _deg_kernel(E, NP):
    ept = E // NW
    nit = ept // K
    sl = NP // NS
    mesh = plsc.VectorSubcoreMesh(core_axis_name="c", subcore_axis_name="s")

    @functools.partial(
        pl.kernel,
        mesh=mesh,
        out_type=jax.ShapeDtypeStruct((NS, NC, NP // NS), jnp.float32),
        scratch_types=[
            pltpu.VMEM((nit, K), jnp.int32),
            pltpu.VMEM((K,), jnp.float32),
            pltpu.VMEM_SHARED((NP,), jnp.float32),
            pltpu.SemaphoreType.DMA,
        ],
    )
    def deg_k(row_hbm, zeros_hbm, out_hbm, idx_v, ones_v, deg_sp, sem):
        c = lax.axis_index("c")
        s = lax.axis_index("s")
        w = c * NS + s
        pltpu.sync_copy(row_hbm.at[w], idx_v)
        pltpu.sync_copy(zeros_hbm, deg_sp.at[pl.ds(s * sl, sl)])
        for i in range(K // 16):
            ones_v[pl.ds(i * 16, 16)] = jnp.full((16,), 1.0, jnp.float32)
        plsc.subcore_barrier()

        pltpu.async_copy(ones_v, deg_sp.at[idx_v.at[0]], sem, add=True)

        def body(it, carry):
            pltpu.async_copy(ones_v, deg_sp.at[idx_v.at[it + 1]], sem, add=True)
            pltpu.make_async_copy(ones_v, deg_sp.at[idx_v.at[it]], sem).wait()
            return carry

        lax.fori_loop(0, nit - 1, body, 0)
        pltpu.make_async_copy(ones_v, deg_sp.at[idx_v.at[nit - 1]], sem).wait()
        plsc.subcore_barrier()
        pltpu.sync_copy(deg_sp.at[pl.ds(s * sl, sl)], out_hbm.at[s, c])

    return deg_k


def _agg_kernel(E, N, NP, D):
    ept = E // NW
    nit = ept // K
    sl = NP // NS
    mesh = plsc.VectorSubcoreMesh(core_axis_name="c", subcore_axis_name="s")

    assert nit % 3 == 2

    @functools.partial(
        pl.kernel,
        mesh=mesh,
        out_type=jax.ShapeDtypeStruct((NC, NP, D), jnp.float32),
        scratch_types=[
            pltpu.VMEM((2, K), jnp.int32),
            pltpu.VMEM((2, K), jnp.int32),
            pltpu.VMEM((2, K), jnp.int32),
            pltpu.VMEM((K,), jnp.int32),
            pltpu.VMEM((K,), jnp.int32),
            pltpu.VMEM((K,), jnp.int32),
            pltpu.VMEM((K, D), jnp.float32),
            pltpu.VMEM((K, D), jnp.float32),
            pltpu.VMEM((K, D), jnp.float32),
            pltpu.VMEM_SHARED((NP, D), jnp.float32),
            pltpu.SemaphoreType.DMA,
            pltpu.SemaphoreType.DMA,
            pltpu.SemaphoreType.DMA,
            pltpu.SemaphoreType.DMA,
            pltpu.SemaphoreType.DMA,
            pltpu.SemaphoreType.DMA,
            pltpu.SemaphoreType.DMA,
            pltpu.SemaphoreType.DMA,
            pltpu.SemaphoreType.DMA,
            pltpu.SemaphoreType.DMA,
        ],
    )
    def agg_k(cr_hbm, xs_hbm, zeros_hbm, out_hbm,
              cr0, cr1, cr2, sb0, sb1, sb2, m0, m1, m2, acc_sp,
              g0, g1, g2, i0, i1, i2, j0, j1, j2, ssem):
        c = lax.axis_index("c")
        s = lax.axis_index("s")
        w = c * NS + s
        pltpu.sync_copy(zeros_hbm, acc_sp.at[pl.ds(s * sl, sl)])
        plsc.subcore_barrier()

        B0, B1, B2 = (cr0, i0), (cr1, i1), (cr2, i2)
        S0, S1, S2 = (sb0, j0), (sb1, j1), (sb2, j2)
        M0, M1, M2 = (m0, g0), (m1, g1), (m2, g2)

        def iload(it, cr):
            pltpu.async_copy(cr_hbm.at[w, it], cr[0], cr[1])

        def iwait(it, cr):
            pltpu.make_async_copy(cr_hbm.at[w, it], cr[0], cr[1]).wait()

        def sload(it, sb):
            pltpu.async_copy(cr_hbm.at[w, it, 0], sb[0], sb[1])

        def swaiti(it, sb):
            pltpu.make_async_copy(cr_hbm.at[w, it, 0], sb[0], sb[1]).wait()

        def gather(it, cr, m):
            pltpu.async_copy(xs_hbm.at[cr[0].at[1]], m[0], m[1])

        def gwait(it, cr, m):
            pltpu.make_async_copy(xs_hbm.at[cr[0].at[1]], m[0], m[1]).wait()

        def scat(sb, m):
            pltpu.async_copy(m[0], acc_sp.at[sb[0]], ssem, add=True)

        def sdrain(sb, m):
            pltpu.make_async_copy(m[0], acc_sp.at[sb[0]], ssem).wait()

        iload(0, B0)
        sload(0, S0)
        iload(1, B1)
        sload(1, S1)
        iload(2, B2)
        iwait(0, B0)
        gather(0, B0, M0)
        iwait(1, B1)
        gather(1, B1, M1)

        def body(t, carry):
            a = 3 * t

            def sub(cc, B, S, M, Bn, Sn, Mn):
                gwait(cc, B, M)
                swaiti(cc, S)
                scat(S, M)

                @pl.when(cc > 0)
                def _():
                    sdrain(S, M)

                @pl.when(cc + 2 < nit)
                def _():
                    sload(cc + 2, Sn)
                    iwait(cc + 2, Bn)
                    gather(cc + 2, Bn, Mn)

                @pl.when(cc + 3 < nit)
                def _():
                    iload(cc + 3, B)

            sub(a, B0, S0, M0, B2, S2, M2)
            sub(a + 1, B1, S1, M1, B0, S0, M0)
            sub(a + 2, B2, S2, M2, B1, S1, M1)
            return carry

        lax.fori_loop(0, (nit - 2) // 3, body, 0)
        gwait(nit - 2, B0, M0)
        swaiti(nit - 2, S0)
        scat(S0, M0)
        sdrain(S0, M0)
        gwait(nit - 1, B1, M1)
        swaiti(nit - 1, S1)
        scat(S1, M1)
        sdrain(S1, M1)
        sdrain(S1, M1)
        plsc.subcore_barrier()
        pltpu.sync_copy(acc_sp.at[pl.ds(s * sl, sl)],
                        out_hbm.at[c, pl.ds(s * sl, sl)])

    return agg_k


def _scale_body(dp_ref, v_ref, o_ref):
    dp = dp_ref[0]
    deg = dp[0] + dp[1]
    dinv = jnp.where(deg > 0.0, lax.rsqrt(deg), 0.0)
    v = v_ref[...]
    agg = v[0] if v.shape[0] == 1 else v[0] + v[1]
    o_ref[...] = agg * dinv[:, None]


def _scale_call(dp3, v, n_out, d):
    nb, _, rb = dp3.shape
    p = v.shape[0]
    return pl.pallas_call(
        _scale_body,
        grid=(nb,),
        in_specs=[
            pl.BlockSpec((1, 2, rb), lambda i: (i, 0, 0)),
            pl.BlockSpec((p, rb, d), lambda i: (0, i, 0)),
        ],
        out_specs=pl.BlockSpec((rb, d), lambda i: (i, 0)),
        out_shape=jax.ShapeDtypeStruct((n_out, d), jnp.float32),
    )(dp3, v)


@jax.jit
def kernel(x, edge_index):
    N, D = x.shape
    E = edge_index.shape[1]
    NP = 10240
    ept = E // NW
    nit = ept // K
    row3 = edge_index[0].reshape(NW, nit, K)
    cr4 = edge_index.reshape(2, NW, nit, K).transpose(1, 2, 0, 3)

    zeros1 = jnp.zeros((NP // NS,), jnp.float32)
    zeros2 = jnp.zeros((NP // NS, D), jnp.float32)

    dp3 = _deg_kernel(E, NP)(row3, zeros1)
    xs = _scale_call(dp3, x.reshape(1, N, D), NP, D)
    parts = _agg_kernel(E, N, NP, D)(cr4, xs, zeros2)
    out = _scale_call(dp3, parts, N, D)
    return out
